# baseline (speedup 1.0000x reference)
import functools

import jax
import jax.numpy as jnp
from jax import lax
from jax.experimental import pallas as pl
from jax.experimental.pallas import tpu as pltpu
from jax.experimental.pallas import tpu_sc as plsc

N = 10000
E = 320000
NC = 2
NS = 16
CHUNK = 100
NCHUNK = E // (NS * CHUNK)
NPAD = 10240
ROWS_PER_TILE = NPAD // NS

_mesh = plsc.VectorSubcoreMesh(core_axis_name="c", subcore_axis_name="s")


@functools.partial(
    pl.kernel,
    mesh=_mesh,
    out_type=jax.ShapeDtypeStruct((NC, NPAD, 16), jnp.float32),
    scratch_types=[
        pltpu.VMEM((NCHUNK, CHUNK), jnp.int32),
        pltpu.VMEM((CHUNK, 16), jnp.float32),
        pltpu.VMEM_SHARED((NPAD, 16), jnp.float32),
    ],
)
def _sc_degree(dst_hbm, ones_hbm, zeros_hbm, out_hbm, dst_v, ones_v, acc_sh):
    c = lax.axis_index("c")
    s = lax.axis_index("s")
    r0 = s * ROWS_PER_TILE
    pltpu.sync_copy(dst_hbm.at[s], dst_v)
    pltpu.sync_copy(ones_hbm, ones_v)
    pltpu.sync_copy(zeros_hbm.at[pl.ds(r0, ROWS_PER_TILE)],
                    acc_sh.at[pl.ds(r0, ROWS_PER_TILE)])
    plsc.subcore_barrier()

    half = NCHUNK // NC

    def body(j, carry):
        pltpu.sync_copy(ones_v, acc_sh.at[dst_v.at[j]], add=True)
        return carry

    lax.fori_loop(c * half, (c + 1) * half, body, 0)
    plsc.subcore_barrier()
    pltpu.sync_copy(acc_sh.at[pl.ds(r0, ROWS_PER_TILE)],
                    out_hbm.at[c, pl.ds(r0, ROWS_PER_TILE)])


def _make_sc_agg(dh):
    @functools.partial(
        pl.kernel,
        mesh=_mesh,
        compiler_params=pltpu.CompilerParams(use_tc_tiling_on_sc=False),
        out_type=jax.ShapeDtypeStruct((NC, NPAD, dh), jnp.float32),
        scratch_types=[
            pltpu.VMEM((NCHUNK, CHUNK), jnp.int32),
            pltpu.VMEM((NCHUNK, CHUNK), jnp.int32),
            [pltpu.VMEM((CHUNK, dh), jnp.float32) for _ in range(4)],
            [pltpu.SemaphoreType.DMA for _ in range(4)],
            [pltpu.SemaphoreType.DMA for _ in range(4)],
            pltpu.VMEM_SHARED((NPAD, dh), jnp.float32),
        ],
    )
    def sc_agg(src_hbm, dst_hbm, hpa_hbm, hpb_hbm, zeros_hbm, out_hbm,
               src_v, dst_v, bufs, gsems, ssems, acc_sh):
        c = lax.axis_index("c")
        s = lax.axis_index("s")
        r0 = s * ROWS_PER_TILE
        pltpu.sync_copy(src_hbm.at[s], src_v)
        pltpu.sync_copy(dst_hbm.at[s], dst_v)
        pltpu.sync_copy(zeros_hbm.at[pl.ds(r0, ROWS_PER_TILE)],
                        acc_sh.at[pl.ds(r0, ROWS_PER_TILE)])
        plsc.subcore_barrier()

        def run(hp_hbm):
            pltpu.async_copy(hp_hbm.at[src_v.at[0]], bufs[0], gsems[0])

            def body(j, carry):
                for p in range(4):
                    q = (p + 1) % 4
                    r = (p + 3) % 4

                    @pl.when(lax.rem(j, 4) == p)
                    def _():
                        pltpu.make_async_copy(hp_hbm.at[src_v.at[j]],
                                              bufs[p], gsems[p]).wait()

                        @pl.when(j >= 1)
                        def _():
                            pltpu.make_async_copy(
                                bufs[r], acc_sh.at[dst_v.at[j - 1]],
                                ssems[r]).wait()

                        pltpu.async_copy(bufs[p], acc_sh.at[dst_v.at[j]],
                                         ssems[p], add=True)

                        @pl.when(j + 1 < NCHUNK)
                        def _():
                            pltpu.async_copy(hp_hbm.at[src_v.at[j + 1]],
                                             bufs[q], gsems[q])
                return carry

            lax.fori_loop(0, NCHUNK, body, 0)
            pltpu.make_async_copy(bufs[(NCHUNK - 1) % 4],
                                  acc_sh.at[dst_v.at[NCHUNK - 1]],
                                  ssems[(NCHUNK - 1) % 4]).wait()

        @pl.when(c == 0)
        def _():
            run(hpa_hbm)

        @pl.when(c == 1)
        def _():
            run(hpb_hbm)

        plsc.subcore_barrier()
        pltpu.sync_copy(acc_sh.at[pl.ds(r0, ROWS_PER_TILE)],
                        out_hbm.at[c, pl.ds(r0, ROWS_PER_TILE)])

    return sc_agg


_sc_agg64 = _make_sc_agg(64)
_sc_agg32 = _make_sc_agg(32)


_BLK = 1000


def _tc_first_body(deg_ref, x_ref, w_ref, hpa_ref, hpb_ref, dis_ref):
    deg = deg_ref[0, :, 0:1] + deg_ref[1, :, 0:1] + 1.0
    dis = lax.rsqrt(deg)
    dis_ref[...] = dis
    hp = jnp.dot(x_ref[...], w_ref[...],
                 preferred_element_type=jnp.float32) * dis
    hpa_ref[...] = hp[:, :64]
    hpb_ref[...] = hp[:, 64:]


def _tc_first(deg, x, w):
    return pl.pallas_call(
        _tc_first_body,
        grid=(N // _BLK,),
        in_specs=[
            pl.BlockSpec((NC, _BLK, 16), lambda i: (0, i, 0)),
            pl.BlockSpec((_BLK, 128), lambda i: (i, 0)),
            pl.BlockSpec((128, 128), lambda i: (0, 0)),
        ],
        out_specs=[
            pl.BlockSpec((_BLK, 64), lambda i: (i, 0)),
            pl.BlockSpec((_BLK, 64), lambda i: (i, 0)),
            pl.BlockSpec((_BLK, 1), lambda i: (i, 0)),
        ],
        out_shape=[
            jax.ShapeDtypeStruct((N, 64), jnp.float32),
            jax.ShapeDtypeStruct((N, 64), jnp.float32),
            jax.ShapeDtypeStruct((N, 1), jnp.float32),
        ],
    )(deg, x, w)


def _tc_mid_body(agg_ref, hpa_ref, hpb_ref, dis_ref, b_ref, w_ref,
                 outa_ref, outb_ref):
    dis = dis_ref[...]
    z = jnp.concatenate(
        [agg_ref[0] + hpa_ref[...], agg_ref[1] + hpb_ref[...]], axis=1)
    z = jnp.maximum(dis * z + b_ref[...], 0.0)
    res = jnp.dot(z, w_ref[...], preferred_element_type=jnp.float32) * dis
    dh = res.shape[1] // 2
    outa_ref[...] = res[:, :dh]
    outb_ref[...] = res[:, dh:]


def _tc_mid(agg, hpa, hpb, dis, b, w):
    dn = w.shape[1]
    dh = agg.shape[2]
    return pl.pallas_call(
        _tc_mid_body,
        grid=(N // _BLK,),
        in_specs=[
            pl.BlockSpec((NC, _BLK, dh), lambda i: (0, i, 0)),
            pl.BlockSpec((_BLK, dh), lambda i: (i, 0)),
            pl.BlockSpec((_BLK, dh), lambda i: (i, 0)),
            pl.BlockSpec((_BLK, 1), lambda i: (i, 0)),
            pl.BlockSpec((1, 128), lambda i: (0, 0)),
            pl.BlockSpec((128, dn), lambda i: (0, 0)),
        ],
        out_specs=[
            pl.BlockSpec((_BLK, dn // 2), lambda i: (i, 0)),
            pl.BlockSpec((_BLK, dn // 2), lambda i: (i, 0)),
        ],
        out_shape=[
            jax.ShapeDtypeStruct((N, dn // 2), jnp.float32),
            jax.ShapeDtypeStruct((N, dn // 2), jnp.float32),
        ],
    )(agg, hpa, hpb, dis, b, w)


def _tc_last_body(agg_ref, hpa_ref, hpb_ref, dis_ref, b_ref, out_ref):
    z = jnp.concatenate(
        [agg_ref[0] + hpa_ref[...], agg_ref[1] + hpb_ref[...]], axis=1)
    out_ref[...] = jnp.maximum(dis_ref[...] * z + b_ref[...], 0.0)


def _tc_last(agg, hpa, hpb, dis, b):
    dh = agg.shape[2]
    return pl.pallas_call(
        _tc_last_body,
        grid=(N // _BLK,),
        in_specs=[
            pl.BlockSpec((NC, _BLK, dh), lambda i: (0, i, 0)),
            pl.BlockSpec((_BLK, dh), lambda i: (i, 0)),
            pl.BlockSpec((_BLK, dh), lambda i: (i, 0)),
            pl.BlockSpec((_BLK, 1), lambda i: (i, 0)),
            pl.BlockSpec((1, 2 * dh), lambda i: (0, 0)),
        ],
        out_specs=pl.BlockSpec((_BLK, 2 * dh), lambda i: (i, 0)),
        out_shape=jax.ShapeDtypeStruct((N, 2 * dh), jnp.float32),
    )(agg, hpa, hpb, dis, b)


def kernel(x, edge_index, W1, b1, W2, b2, W3, b3):
    src = edge_index[0].reshape(NS, NCHUNK, CHUNK)
    dst = edge_index[1].reshape(NS, NCHUNK, CHUNK)
    ones16 = jnp.ones((CHUNK, 16), jnp.float32)
    zeros16 = jnp.zeros((NPAD, 16), jnp.float32)
    zeros64 = jnp.zeros((NPAD, 64), jnp.float32)
    zeros32 = jnp.zeros((NPAD, 32), jnp.float32)

    deg = _sc_degree(dst, ones16, zeros16)
    hpa1, hpb1, dis = _tc_first(deg, x, W1)
    agg1 = _sc_agg64(src, dst, hpa1, hpb1, zeros64)
    hpa2, hpb2 = _tc_mid(agg1, hpa1, hpb1, dis, b1.reshape(1, -1), W2)
    agg2 = _sc_agg64(src, dst, hpa2, hpb2, zeros64)
    hpa3, hpb3 = _tc_mid(agg2, hpa2, hpb2, dis, b2.reshape(1, -1), W3)
    agg3 = _sc_agg32(src, dst, hpa3, hpb3, zeros32)
    return _tc_last(agg3, hpa3, hpb3, dis, b3.reshape(1, -1))


# speedup vs baseline: 1.0207x; 1.0207x over previous
import functools

import jax
import jax.numpy as jnp
from jax import lax
from jax.experimental import pallas as pl
from jax.experimental.pallas import tpu as pltpu
from jax.experimental.pallas import tpu_sc as plsc

N = 10000
E = 320000
NC = 2
NS = 16
CHUNK = 100
NCHUNK = E // (NS * CHUNK)
NPAD = 10240
ROWS_PER_TILE = NPAD // NS

_mesh = plsc.VectorSubcoreMesh(core_axis_name="c", subcore_axis_name="s")


@functools.partial(
    pl.kernel,
    mesh=_mesh,
    compiler_params=pltpu.CompilerParams(use_tc_tiling_on_sc=False),
    out_type=jax.ShapeDtypeStruct((NC, NPAD, 16), jnp.float32),
    scratch_types=[
        pltpu.VMEM((NCHUNK, CHUNK), jnp.int32),
        pltpu.VMEM((CHUNK, 16), jnp.float32),
        pltpu.VMEM_SHARED((NPAD, 16), jnp.float32),
    ],
)
def _sc_degree(dst_hbm, ones_hbm, zeros_hbm, out_hbm, dst_v, ones_v, acc_sh):
    c = lax.axis_index("c")
    s = lax.axis_index("s")
    r0 = s * ROWS_PER_TILE
    pltpu.sync_copy(dst_hbm.at[s], dst_v)
    pltpu.sync_copy(ones_hbm, ones_v)
    pltpu.sync_copy(zeros_hbm.at[pl.ds(r0, ROWS_PER_TILE)],
                    acc_sh.at[pl.ds(r0, ROWS_PER_TILE)])
    plsc.subcore_barrier()

    half = NCHUNK // NC

    def body(j, carry):
        pltpu.sync_copy(ones_v, acc_sh.at[dst_v.at[j]], add=True)
        return carry

    lax.fori_loop(c * half, (c + 1) * half, body, 0)
    plsc.subcore_barrier()
    pltpu.sync_copy(acc_sh.at[pl.ds(r0, ROWS_PER_TILE)],
                    out_hbm.at[c, pl.ds(r0, ROWS_PER_TILE)])


def _make_sc_agg(dh):
    @functools.partial(
        pl.kernel,
        mesh=_mesh,
        compiler_params=pltpu.CompilerParams(use_tc_tiling_on_sc=False),
        out_type=jax.ShapeDtypeStruct((NC, NPAD, dh), jnp.float32),
        scratch_types=[
            pltpu.VMEM((NCHUNK, CHUNK), jnp.int32),
            pltpu.VMEM((NCHUNK, CHUNK), jnp.int32),
            [pltpu.VMEM((CHUNK, dh), jnp.float32) for _ in range(4)],
            [pltpu.SemaphoreType.DMA for _ in range(4)],
            [pltpu.SemaphoreType.DMA for _ in range(4)],
            pltpu.VMEM_SHARED((NPAD, dh), jnp.float32),
        ],
    )
    def sc_agg(src_hbm, dst_hbm, hpa_hbm, hpb_hbm, out_hbm,
               src_v, dst_v, bufs, gsems, ssems, acc_sh):
        c = lax.axis_index("c")
        s = lax.axis_index("s")
        r0 = s * ROWS_PER_TILE
        pltpu.sync_copy(src_hbm.at[s], src_v)
        pltpu.sync_copy(dst_hbm.at[s], dst_v)

        def seed(hp_hbm):
            @pl.when(s < NS - 1)
            def _():
                pltpu.sync_copy(hp_hbm.at[pl.ds(r0, ROWS_PER_TILE)],
                                acc_sh.at[pl.ds(r0, ROWS_PER_TILE)])

            @pl.when(s == NS - 1)
            def _():
                tail = N - (NS - 1) * ROWS_PER_TILE
                pltpu.sync_copy(hp_hbm.at[pl.ds(r0, tail)],
                                acc_sh.at[pl.ds(r0, tail)])

        @pl.when(c == 0)
        def _():
            seed(hpa_hbm)

        @pl.when(c == 1)
        def _():
            seed(hpb_hbm)

        plsc.subcore_barrier()

        def run(hp_hbm):
            pltpu.async_copy(hp_hbm.at[src_v.at[0]], bufs[0], gsems[0])

            def body(j, carry):
                for p in range(4):
                    q = (p + 1) % 4
                    r = (p + 3) % 4

                    @pl.when(lax.rem(j, 4) == p)
                    def _():
                        pltpu.make_async_copy(hp_hbm.at[src_v.at[j]],
                                              bufs[p], gsems[p]).wait()

                        @pl.when(j >= 1)
                        def _():
                            pltpu.make_async_copy(
                                bufs[r], acc_sh.at[dst_v.at[j - 1]],
                                ssems[r]).wait()

                        pltpu.async_copy(bufs[p], acc_sh.at[dst_v.at[j]],
                                         ssems[p], add=True)

                        @pl.when(j + 1 < NCHUNK)
                        def _():
                            pltpu.async_copy(hp_hbm.at[src_v.at[j + 1]],
                                             bufs[q], gsems[q])
                return carry

            lax.fori_loop(0, NCHUNK, body, 0)
            pltpu.make_async_copy(bufs[(NCHUNK - 1) % 4],
                                  acc_sh.at[dst_v.at[NCHUNK - 1]],
                                  ssems[(NCHUNK - 1) % 4]).wait()

        @pl.when(c == 0)
        def _():
            run(hpa_hbm)

        @pl.when(c == 1)
        def _():
            run(hpb_hbm)

        plsc.subcore_barrier()
        pltpu.sync_copy(acc_sh.at[pl.ds(r0, ROWS_PER_TILE)],
                        out_hbm.at[c, pl.ds(r0, ROWS_PER_TILE)])

    return sc_agg


_sc_agg64 = _make_sc_agg(64)
_sc_agg32 = _make_sc_agg(32)


_BLK = 1000


def _tc_first_body(deg_ref, x_ref, w_ref, hpa_ref, hpb_ref, dis_ref):
    deg = deg_ref[0, :, 0:1] + deg_ref[1, :, 0:1] + 1.0
    dis = lax.rsqrt(deg)
    dis_ref[...] = dis
    hp = jnp.dot(x_ref[...], w_ref[...],
                 preferred_element_type=jnp.float32) * dis
    hpa_ref[...] = hp[:, :64]
    hpb_ref[...] = hp[:, 64:]


def _tc_first(deg, x, w):
    return pl.pallas_call(
        _tc_first_body,
        grid=(N // _BLK,),
        in_specs=[
            pl.BlockSpec((NC, _BLK, 16), lambda i: (0, i, 0)),
            pl.BlockSpec((_BLK, 128), lambda i: (i, 0)),
            pl.BlockSpec((128, 128), lambda i: (0, 0)),
        ],
        out_specs=[
            pl.BlockSpec((_BLK, 64), lambda i: (i, 0)),
            pl.BlockSpec((_BLK, 64), lambda i: (i, 0)),
            pl.BlockSpec((_BLK, 1), lambda i: (i, 0)),
        ],
        out_shape=[
            jax.ShapeDtypeStruct((N, 64), jnp.float32),
            jax.ShapeDtypeStruct((N, 64), jnp.float32),
            jax.ShapeDtypeStruct((N, 1), jnp.float32),
        ],
    )(deg, x, w)


def _tc_mid_body(agg_ref, dis_ref, b_ref, w_ref, outa_ref, outb_ref):
    dis = dis_ref[...]
    z = jnp.concatenate([agg_ref[0], agg_ref[1]], axis=1)
    z = jnp.maximum(dis * z + b_ref[...], 0.0)
    res = jnp.dot(z, w_ref[...], preferred_element_type=jnp.float32) * dis
    dh = res.shape[1] // 2
    outa_ref[...] = res[:, :dh]
    outb_ref[...] = res[:, dh:]


def _tc_mid(agg, dis, b, w):
    dn = w.shape[1]
    dh = agg.shape[2]
    return pl.pallas_call(
        _tc_mid_body,
        grid=(N // _BLK,),
        in_specs=[
            pl.BlockSpec((NC, _BLK, dh), lambda i: (0, i, 0)),
            pl.BlockSpec((_BLK, 1), lambda i: (i, 0)),
            pl.BlockSpec((1, 128), lambda i: (0, 0)),
            pl.BlockSpec((128, dn), lambda i: (0, 0)),
        ],
        out_specs=[
            pl.BlockSpec((_BLK, dn // 2), lambda i: (i, 0)),
            pl.BlockSpec((_BLK, dn // 2), lambda i: (i, 0)),
        ],
        out_shape=[
            jax.ShapeDtypeStruct((N, dn // 2), jnp.float32),
            jax.ShapeDtypeStruct((N, dn // 2), jnp.float32),
        ],
    )(agg, dis, b, w)


def _tc_last_body(agg_ref, dis_ref, b_ref, out_ref):
    z = jnp.concatenate([agg_ref[0], agg_ref[1]], axis=1)
    out_ref[...] = jnp.maximum(dis_ref[...] * z + b_ref[...], 0.0)


def _tc_last(agg, dis, b):
    dh = agg.shape[2]
    return pl.pallas_call(
        _tc_last_body,
        grid=(N // _BLK,),
        in_specs=[
            pl.BlockSpec((NC, _BLK, dh), lambda i: (0, i, 0)),
            pl.BlockSpec((_BLK, 1), lambda i: (i, 0)),
            pl.BlockSpec((1, 2 * dh), lambda i: (0, 0)),
        ],
        out_specs=pl.BlockSpec((_BLK, 2 * dh), lambda i: (i, 0)),
        out_shape=jax.ShapeDtypeStruct((N, 2 * dh), jnp.float32),
    )(agg, dis, b)


def kernel(x, edge_index, W1, b1, W2, b2, W3, b3):
    src = edge_index[0].reshape(NS, NCHUNK, CHUNK)
    dst = edge_index[1].reshape(NS, NCHUNK, CHUNK)
    ones16 = jnp.ones((CHUNK, 16), jnp.float32)
    zeros16 = jnp.zeros((NPAD, 16), jnp.float32)

    deg = _sc_degree(dst, ones16, zeros16)
    hpa1, hpb1, dis = _tc_first(deg, x, W1)
    agg1 = _sc_agg64(src, dst, hpa1, hpb1)
    hpa2, hpb2 = _tc_mid(agg1, dis, b1.reshape(1, -1), W2)
    agg2 = _sc_agg64(src, dst, hpa2, hpb2)
    hpa3, hpb3 = _tc_mid(agg2, dis, b2.reshape(1, -1), W3)
    agg3 = _sc_agg32(src, dst, hpa3, hpb3)
    return _tc_last(agg3, dis, b3.reshape(1, -1))


# speedup vs baseline: 1.0231x; 1.0023x over previous
import functools

import jax
import jax.numpy as jnp
from jax import lax
from jax.experimental import pallas as pl
from jax.experimental.pallas import tpu as pltpu
from jax.experimental.pallas import tpu_sc as plsc

N = 10000
E = 320000
NC = 2
NS = 16
CHUNK = 100
NCHUNK = E // (NS * CHUNK)
NPAD = 10240
ROWS_PER_TILE = NPAD // NS

_mesh = plsc.VectorSubcoreMesh(core_axis_name="c", subcore_axis_name="s")


@functools.partial(
    pl.kernel,
    mesh=_mesh,
    compiler_params=pltpu.CompilerParams(use_tc_tiling_on_sc=False),
    out_type=jax.ShapeDtypeStruct((NC, NPAD, 16), jnp.float32),
    scratch_types=[
        pltpu.VMEM((NCHUNK, CHUNK), jnp.int32),
        pltpu.VMEM((CHUNK, 16), jnp.float32),
        pltpu.VMEM_SHARED((NPAD, 16), jnp.float32),
    ],
)
def _sc_degree(edge_hbm, ones_hbm, zeros_hbm, out_hbm, dst_v, ones_v, acc_sh):
    c = lax.axis_index("c")
    s = lax.axis_index("s")
    r0 = s * ROWS_PER_TILE
    pltpu.sync_copy(edge_hbm.at[1, s], dst_v)
    pltpu.sync_copy(ones_hbm, ones_v)
    pltpu.sync_copy(zeros_hbm.at[pl.ds(r0, ROWS_PER_TILE)],
                    acc_sh.at[pl.ds(r0, ROWS_PER_TILE)])
    plsc.subcore_barrier()

    half = NCHUNK // NC

    def body(j, carry):
        pltpu.sync_copy(ones_v, acc_sh.at[dst_v.at[j]], add=True)
        return carry

    lax.fori_loop(c * half, (c + 1) * half, body, 0)
    plsc.subcore_barrier()
    pltpu.sync_copy(acc_sh.at[pl.ds(r0, ROWS_PER_TILE)],
                    out_hbm.at[c, pl.ds(r0, ROWS_PER_TILE)])


def _make_sc_agg(dh):
    @functools.partial(
        pl.kernel,
        mesh=_mesh,
        compiler_params=pltpu.CompilerParams(use_tc_tiling_on_sc=False),
        out_type=jax.ShapeDtypeStruct((NC, NPAD, dh), jnp.float32),
        scratch_types=[
            pltpu.VMEM((NCHUNK, CHUNK), jnp.int32),
            pltpu.VMEM((NCHUNK, CHUNK), jnp.int32),
            [pltpu.VMEM((CHUNK, dh), jnp.float32) for _ in range(4)],
            [pltpu.SemaphoreType.DMA for _ in range(4)],
            [pltpu.SemaphoreType.DMA for _ in range(4)],
            pltpu.VMEM_SHARED((NPAD, dh), jnp.float32),
        ],
    )
    def sc_agg(edge_hbm, hpa_hbm, hpb_hbm, out_hbm,
               src_v, dst_v, bufs, gsems, ssems, acc_sh):
        c = lax.axis_index("c")
        s = lax.axis_index("s")
        r0 = s * ROWS_PER_TILE
        pltpu.sync_copy(edge_hbm.at[0, s], src_v)
        pltpu.sync_copy(edge_hbm.at[1, s], dst_v)

        def seed(hp_hbm):
            @pl.when(s < NS - 1)
            def _():
                pltpu.sync_copy(hp_hbm.at[pl.ds(r0, ROWS_PER_TILE)],
                                acc_sh.at[pl.ds(r0, ROWS_PER_TILE)])

            @pl.when(s == NS - 1)
            def _():
                tail = N - (NS - 1) * ROWS_PER_TILE
                pltpu.sync_copy(hp_hbm.at[pl.ds(r0, tail)],
                                acc_sh.at[pl.ds(r0, tail)])

        @pl.when(c == 0)
        def _():
            seed(hpa_hbm)

        @pl.when(c == 1)
        def _():
            seed(hpb_hbm)

        plsc.subcore_barrier()

        def run(hp_hbm):
            pltpu.async_copy(hp_hbm.at[src_v.at[0]], bufs[0], gsems[0])

            def body(j, carry):
                for p in range(4):
                    q = (p + 1) % 4
                    r = (p + 3) % 4

                    @pl.when(lax.rem(j, 4) == p)
                    def _():
                        pltpu.make_async_copy(hp_hbm.at[src_v.at[j]],
                                              bufs[p], gsems[p]).wait()

                        @pl.when(j >= 1)
                        def _():
                            pltpu.make_async_copy(
                                bufs[r], acc_sh.at[dst_v.at[j - 1]],
                                ssems[r]).wait()

                        pltpu.async_copy(bufs[p], acc_sh.at[dst_v.at[j]],
                                         ssems[p], add=True)

                        @pl.when(j + 1 < NCHUNK)
                        def _():
                            pltpu.async_copy(hp_hbm.at[src_v.at[j + 1]],
                                             bufs[q], gsems[q])
                return carry

            lax.fori_loop(0, NCHUNK, body, 0)
            pltpu.make_async_copy(bufs[(NCHUNK - 1) % 4],
                                  acc_sh.at[dst_v.at[NCHUNK - 1]],
                                  ssems[(NCHUNK - 1) % 4]).wait()

        @pl.when(c == 0)
        def _():
            run(hpa_hbm)

        @pl.when(c == 1)
        def _():
            run(hpb_hbm)

        plsc.subcore_barrier()
        pltpu.sync_copy(acc_sh.at[pl.ds(r0, ROWS_PER_TILE)],
                        out_hbm.at[c, pl.ds(r0, ROWS_PER_TILE)])

    return sc_agg


_sc_agg64 = _make_sc_agg(64)
_sc_agg32 = _make_sc_agg(32)


_BLK = 1000


def _tc_first_body(deg_ref, x_ref, w_ref, hpa_ref, hpb_ref, dis_ref):
    deg = deg_ref[0, :, 0:1] + deg_ref[1, :, 0:1] + 1.0
    dis = lax.rsqrt(deg)
    dis_ref[...] = dis
    hp = jnp.dot(x_ref[...], w_ref[...],
                 preferred_element_type=jnp.float32) * dis
    hpa_ref[...] = hp[:, :64]
    hpb_ref[...] = hp[:, 64:]


def _tc_first(deg, x, w):
    return pl.pallas_call(
        _tc_first_body,
        grid=(N // _BLK,),
        in_specs=[
            pl.BlockSpec((NC, _BLK, 16), lambda i: (0, i, 0)),
            pl.BlockSpec((_BLK, 128), lambda i: (i, 0)),
            pl.BlockSpec((128, 128), lambda i: (0, 0)),
        ],
        out_specs=[
            pl.BlockSpec((_BLK, 64), lambda i: (i, 0)),
            pl.BlockSpec((_BLK, 64), lambda i: (i, 0)),
            pl.BlockSpec((_BLK, 1), lambda i: (i, 0)),
        ],
        out_shape=[
            jax.ShapeDtypeStruct((N, 64), jnp.float32),
            jax.ShapeDtypeStruct((N, 64), jnp.float32),
            jax.ShapeDtypeStruct((N, 1), jnp.float32),
        ],
    )(deg, x, w)


def _tc_mid_body(agg_ref, dis_ref, b_ref, w_ref, outa_ref, outb_ref):
    dis = dis_ref[...]
    z = jnp.concatenate([agg_ref[0], agg_ref[1]], axis=1)
    z = jnp.maximum(dis * z + b_ref[...], 0.0)
    res = jnp.dot(z, w_ref[...], preferred_element_type=jnp.float32) * dis
    dh = res.shape[1] // 2
    outa_ref[...] = res[:, :dh]
    outb_ref[...] = res[:, dh:]


def _tc_mid(agg, dis, b, w):
    dn = w.shape[1]
    dh = agg.shape[2]
    return pl.pallas_call(
        _tc_mid_body,
        grid=(N // _BLK,),
        in_specs=[
            pl.BlockSpec((NC, _BLK, dh), lambda i: (0, i, 0)),
            pl.BlockSpec((_BLK, 1), lambda i: (i, 0)),
            pl.BlockSpec((1, 128), lambda i: (0, 0)),
            pl.BlockSpec((128, dn), lambda i: (0, 0)),
        ],
        out_specs=[
            pl.BlockSpec((_BLK, dn // 2), lambda i: (i, 0)),
            pl.BlockSpec((_BLK, dn // 2), lambda i: (i, 0)),
        ],
        out_shape=[
            jax.ShapeDtypeStruct((N, dn // 2), jnp.float32),
            jax.ShapeDtypeStruct((N, dn // 2), jnp.float32),
        ],
    )(agg, dis, b, w)


def _tc_last_body(agg_ref, dis_ref, b_ref, out_ref):
    z = jnp.concatenate([agg_ref[0], agg_ref[1]], axis=1)
    out_ref[...] = jnp.maximum(dis_ref[...] * z + b_ref[...], 0.0)


def _tc_last(agg, dis, b):
    dh = agg.shape[2]
    return pl.pallas_call(
        _tc_last_body,
        grid=(N // _BLK,),
        in_specs=[
            pl.BlockSpec((NC, _BLK, dh), lambda i: (0, i, 0)),
            pl.BlockSpec((_BLK, 1), lambda i: (i, 0)),
            pl.BlockSpec((1, 2 * dh), lambda i: (0, 0)),
        ],
        out_specs=pl.BlockSpec((_BLK, 2 * dh), lambda i: (i, 0)),
        out_shape=jax.ShapeDtypeStruct((N, 2 * dh), jnp.float32),
    )(agg, dis, b)


def kernel(x, edge_index, W1, b1, W2, b2, W3, b3):
    edges = edge_index.reshape(2, NS, NCHUNK, CHUNK)
    ones16 = jnp.ones((CHUNK, 16), jnp.float32)
    zeros16 = jnp.zeros((NPAD, 16), jnp.float32)

    deg = _sc_degree(edges, ones16, zeros16)
    hpa1, hpb1, dis = _tc_first(deg, x, W1)
    agg1 = _sc_agg64(edges, hpa1, hpb1)
    hpa2, hpb2 = _tc_mid(agg1, dis, b1.reshape(1, -1), W2)
    agg2 = _sc_agg64(edges, hpa2, hpb2)
    hpa3, hpb3 = _tc_mid(agg2, dis, b2.reshape(1, -1), W3)
    agg3 = _sc_agg32(edges, hpa3, hpb3)
    return _tc_last(agg3, dis, b3.reshape(1, -1))
